# baseline (speedup 1.0000x reference)
import jax
import jax.numpy as jnp
from jax.experimental import pallas as pl
from jax.experimental.pallas import tpu as pltpu
import jax.experimental.pallas.tpu_sc as plsc

_MIDI_PITCHES = 88
_MIN_MIDI_PITCH = 21
_C = 11
_HOP = 512
_TW = 512
_N = 256


def _sc_body(pitch_hbm, start_hbm, end_hbm, tpt_hbm, out_hbm,
             pitch_s, start_s, end_s, tpt_s, buf, sem):
    core = jax.lax.axis_index("c")
    sub = jax.lax.axis_index("s")
    k = core * 16 + sub
    bidx = k // 4
    t0 = (k % 4) * _TW

    for src, dst in ((pitch_hbm, pitch_s), (start_hbm, start_s),
                     (end_hbm, end_s), (tpt_hbm, tpt_s)):
        cp = pltpu.make_async_copy(src.at[bidx], dst, sem)
        cp.start()
        cp.wait()

    lane = jax.lax.broadcasted_iota(jnp.int32, (16,), 0)

    for c in range(_C):
        def _zero_row(r, _):
            for j in range(_TW // 16):
                buf[r, pl.ds(j * 16, 16)] = jnp.zeros((16,), jnp.float32)
            return 0
        jax.lax.fori_loop(0, _MIDI_PITCHES, _zero_row, 0)

        def _group(g, _):
            o16 = g * 16
            vp = pitch_s[pl.ds(o16, 16)]
            vs = start_s[pl.ds(o16, 16)]
            ve = end_s[pl.ds(o16, 16)]
            vv = tpt_s[c, pl.ds(o16, 16)]
            for u in range(16):
                p = vp[u]
                ts = jnp.clip(vs[u] - t0, 0, _TW)
                te = jnp.clip(ve[u] - t0, 0, _TW)
                vval = jnp.full((16,), vv[u], jnp.float32)

                def _chunk(j, _):
                    o = j * 16
                    tt = o + lane
                    xm = jnp.where((tt >= ts) & (tt < te), vval, 0.0)
                    plsc.addupdate(buf.at[p, pl.ds(o, 16)], xm)
                    return 0
                jax.lax.fori_loop(ts // 16, (te + 15) // 16, _chunk, 0)
            return 0
        jax.lax.fori_loop(0, _N // 16, _group, 0)

        cpo = pltpu.make_async_copy(
            buf, out_hbm.at[bidx, c, :, pl.ds(t0, _TW)], sem)
        cpo.start()
        cpo.wait()


def kernel(note_croppings, timbre_probs, pianorolls):
    b, n, _ = note_croppings.shape
    t_frames = pianorolls.shape[1]
    pitch = note_croppings[:, :, 0] - _MIN_MIDI_PITCH
    start_idx = note_croppings[:, :, 1] // _HOP
    end_raw = note_croppings[:, :, 2]
    end_idx = jnp.where(end_raw >= 0, end_raw // _HOP, -1)
    tpt = timbre_probs.transpose(0, 2, 1)
    f = pl.kernel(
        _sc_body,
        out_type=jax.ShapeDtypeStruct((b, _C, _MIDI_PITCHES, t_frames),
                                      jnp.float32),
        mesh=plsc.VectorSubcoreMesh(core_axis_name="c",
                                    subcore_axis_name="s"),
        scratch_types=[
            pltpu.VMEM((n,), jnp.int32),
            pltpu.VMEM((n,), jnp.int32),
            pltpu.VMEM((n,), jnp.int32),
            pltpu.VMEM((_C, n), jnp.float32),
            pltpu.VMEM((_MIDI_PITCHES, _TW), jnp.float32),
            pltpu.SemaphoreType.DMA,
        ],
    )
    out = f(pitch, start_idx, end_idx, tpt)
    return out.transpose(0, 3, 2, 1)


# speedup vs baseline: 5.7192x; 5.7192x over previous
import jax
import jax.numpy as jnp
from jax.experimental import pallas as pl
from jax.experimental.pallas import tpu as pltpu

_MIDI_PITCHES = 88
_MIN_MIDI_PITCH = 21
_C = 11
_HOP_SHIFT = 9
_PC = _MIDI_PITCHES * _C


def _body(nc_ref, tp_ref, out_ref):
    n = nc_ref.shape[1]
    t_frames = out_ref.shape[3]
    nc = nc_ref[0]
    tp = tp_ref[0]

    pitch_col = nc[:, 0:1] - _MIN_MIDI_PITCH
    start_col = jnp.right_shift(nc[:, 1:2], _HOP_SHIFT)
    end_raw = nc[:, 2:3]
    end_col = jnp.where(end_raw >= 0,
                        jnp.right_shift(end_raw, _HOP_SHIFT), -1)

    tg = jax.lax.broadcasted_iota(jnp.int32, (n, t_frames), 1)
    mask = ((tg >= start_col) & (tg < end_col)).astype(jnp.float32)

    q_row = jax.lax.broadcasted_iota(jnp.int32, (1, _PC), 1)
    pm = (q_row % _MIDI_PITCHES == pitch_col).astype(jnp.float32)
    s_sel = (jax.lax.broadcasted_iota(jnp.int32, (_C, _PC), 0)
             == jax.lax.broadcasted_iota(jnp.int32, (_C, _PC), 1)
             // _MIDI_PITCHES).astype(jnp.float32)
    tpsel = jnp.dot(tp, s_sel, preferred_element_type=jnp.float32)
    m_mat = pm * tpsel

    res = jax.lax.dot_general(m_mat, mask, (((0,), (0,)), ((), ())),
                              preferred_element_type=jnp.float32)
    out_ref[0] = res.reshape(_C, _MIDI_PITCHES, t_frames)


def kernel(note_croppings, timbre_probs, pianorolls):
    b, n, _ = note_croppings.shape
    t_frames = pianorolls.shape[1]
    out = pl.pallas_call(
        _body,
        grid=(b,),
        in_specs=[
            pl.BlockSpec((1, n, 3), lambda i: (i, 0, 0)),
            pl.BlockSpec((1, n, _C), lambda i: (i, 0, 0)),
        ],
        out_specs=pl.BlockSpec((1, _C, _MIDI_PITCHES, t_frames),
                               lambda i: (i, 0, 0, 0)),
        out_shape=jax.ShapeDtypeStruct((b, _C, _MIDI_PITCHES, t_frames),
                                       jnp.float32),
        compiler_params=pltpu.CompilerParams(
            dimension_semantics=("parallel",)),
    )(note_croppings, timbre_probs)
    return out.transpose(0, 3, 2, 1)
